# baseline (speedup 1.0000x reference)
import functools

import jax
import jax.numpy as jnp
from jax import lax
from jax.experimental import pallas as pl
from jax.experimental.pallas import tpu as pltpu
from jax.experimental.pallas import tpu_sc as plsc

_FEAT = 64
_C = 100000
_B = 1024

_LAMB = max(5.0, 1500.0 / 1.1)
_COEF = 1.0 / (1.0 + _LAMB)
_C1 = 0.7071067811865476


def _sc_gather(w_flat, target_2d):
    info = plsc.get_sparse_core_info()
    nc, ns = info.num_cores, info.num_subcores
    nw = nc * ns
    rows_per_w = _FEAT // nw
    n_chunk = _B // 128
    mesh = plsc.VectorSubcoreMesh(core_axis_name="c", subcore_axis_name="s")

    @functools.partial(
        pl.kernel,
        mesh=mesh,
        out_type=jax.ShapeDtypeStruct((_FEAT, _B), jnp.float32),
        scratch_types=[
            pltpu.VMEM((n_chunk, 128), jnp.int32),
            pltpu.VMEM((rows_per_w, _B), jnp.float32),
            pltpu.SemaphoreType.DMA,
        ],
    )
    def gather_kernel(w_hbm, tgt_hbm, out_hbm, tgt_v, dat_v, sem):
        wid = lax.axis_index("s") * nc + lax.axis_index("c")
        d0 = wid * rows_per_w
        pltpu.sync_copy(tgt_hbm, tgt_v)
        copies = []
        for r in range(rows_per_w):
            row = w_hbm.at[pl.ds((d0 + r) * _C, _C)]
            for j in range(n_chunk):
                copies.append(pltpu.async_copy(
                    row.at[tgt_v.at[j]],
                    dat_v.at[r, pl.ds(j * 128, 128)],
                    sem))
        for c in copies:
            c.wait()
        pltpu.sync_copy(dat_v, out_hbm.at[pl.ds(d0, rows_per_w)])

    return gather_kernel(w_flat, target_2d)


def _tc_loss_body(x_ref, wg_ref, tcol_ref, trow_ref, out_ref):
    xv = x_ref[...]
    wg = wg_ref[...]
    xlen2 = jnp.sum(xv * xv, axis=1, keepdims=True)
    wn2 = jnp.sum(wg * wg, axis=0, keepdims=True)
    wgn = wg * lax.rsqrt(wn2)
    obase = jnp.dot(xv, wgn, preferred_element_type=jnp.float32,
                    precision=lax.Precision.HIGHEST)
    wgnt = wgn.T
    diag = jnp.sum(xv * wgnt, axis=1, keepdims=True)
    xlen = jnp.sqrt(xlen2)
    ct = jnp.clip(diag * lax.rsqrt(xlen2), -1.0, 1.0)
    ct2 = ct * ct
    cm = 8.0 * ct2 * ct2 - 8.0 * ct2 + 1.0
    kf = ((ct < _C1).astype(jnp.float32)
          + (ct < 0.0).astype(jnp.float32)
          + (ct < -_C1).astype(jnp.float32)
          + (ct <= -1.0).astype(jnp.float32))
    sign = jnp.where((kf == 1.0) | (kf == 3.0), -1.0, 1.0)
    phi = sign * cm - 2.0 * kf
    base = ct * xlen
    v = base + _COEF * (phi * xlen - base)
    mask = tcol_ref[...] == trow_ref[...]
    o = jnp.where(mask, v, obase)
    m = jnp.max(o, axis=0, keepdims=True)
    lse = m + jnp.log(jnp.sum(jnp.exp(o - m), axis=0, keepdims=True))
    loss = (jnp.sum(lse) - jnp.sum(v)) * (1.0 / _B)
    out_ref[...] = jnp.reshape(loss, (1, 1))


def _tc_loss(x, wg, t_col, t_row, interpret=False):
    return pl.pallas_call(
        _tc_loss_body,
        out_shape=jax.ShapeDtypeStruct((1, 1), jnp.float32),
        interpret=interpret,
    )(x, wg, t_col, t_row)


def kernel(input, target, W):
    w_flat = jnp.concatenate(
        [W[k * 8:(k + 1) * 8].reshape(-1) for k in range(_FEAT // 8)])
    wg = _sc_gather(w_flat, target.reshape(_B // 128, 128))
    t_col = target.reshape(_B, 1)
    t_row = target.reshape(1, _B)
    loss = _tc_loss(input, wg, t_col, t_row)
    return loss[0, 0]


# speedup vs baseline: 4.9004x; 4.9004x over previous
import functools

import jax
import jax.numpy as jnp
from jax import lax
from jax.experimental import pallas as pl
from jax.experimental.pallas import tpu as pltpu
from jax.experimental.pallas import tpu_sc as plsc

_FEAT = 64
_C = 100000
_B = 1024

_LAMB = max(5.0, 1500.0 / 1.1)
_COEF = 1.0 / (1.0 + _LAMB)
_C1 = 0.7071067811865476


def _sc_gather(w_flat, target_2d):
    info = plsc.get_sparse_core_info()
    nc, ns = info.num_cores, info.num_subcores
    nw = nc * ns
    rows_per_w = _FEAT // nw
    n_chunk = _B // 128
    mesh = plsc.VectorSubcoreMesh(core_axis_name="c", subcore_axis_name="s")

    @functools.partial(
        pl.kernel,
        mesh=mesh,
        out_type=jax.ShapeDtypeStruct((_FEAT, _B), jnp.float32),
        scratch_types=[
            pltpu.VMEM((n_chunk, 128), jnp.int32),
            pltpu.VMEM((rows_per_w, _B), jnp.float32),
            pltpu.SemaphoreType.DMA,
        ],
    )
    def gather_kernel(w_hbm, tgt_hbm, out_hbm, tgt_v, dat_v, sem):
        wid = lax.axis_index("s") * nc + lax.axis_index("c")
        d0 = wid * rows_per_w
        pltpu.sync_copy(tgt_hbm, tgt_v)
        copies = []
        for r in range(rows_per_w):
            row = w_hbm.at[pl.ds((d0 + r) * _C, _C)]
            for j in range(n_chunk):
                copies.append(pltpu.async_copy(
                    row.at[tgt_v.at[j]],
                    dat_v.at[r, pl.ds(j * 128, 128)],
                    sem))
        for c in copies:
            c.wait()
        pltpu.sync_copy(dat_v, out_hbm.at[pl.ds(d0, rows_per_w)])

    return gather_kernel(w_flat, target_2d)


def _tc_loss_body(x_ref, wg_ref, tcol_ref, trow_ref, out_ref):
    xv = x_ref[...]
    wg = wg_ref[...]
    xlen2 = jnp.sum(xv * xv, axis=1, keepdims=True)
    wn2 = jnp.sum(wg * wg, axis=0, keepdims=True)
    wgn = wg * lax.rsqrt(wn2)
    obase = jnp.dot(xv, wgn, preferred_element_type=jnp.float32,
                    precision=lax.Precision.DEFAULT)
    wgnt = wgn.T
    diag = jnp.sum(xv * wgnt, axis=1, keepdims=True)
    xlen = jnp.sqrt(xlen2)
    ct = jnp.clip(diag * lax.rsqrt(xlen2), -1.0, 1.0)
    ct2 = ct * ct
    cm = 8.0 * ct2 * ct2 - 8.0 * ct2 + 1.0
    kf = ((ct < _C1).astype(jnp.float32)
          + (ct < 0.0).astype(jnp.float32)
          + (ct < -_C1).astype(jnp.float32)
          + (ct <= -1.0).astype(jnp.float32))
    sign = jnp.where((kf == 1.0) | (kf == 3.0), -1.0, 1.0)
    phi = sign * cm - 2.0 * kf
    base = ct * xlen
    v = base + _COEF * (phi * xlen - base)
    mask = tcol_ref[...] == trow_ref[...]
    o = jnp.where(mask, v, obase)
    m = jnp.max(o, axis=0, keepdims=True)
    lse = m + jnp.log(jnp.sum(jnp.exp(o - m), axis=0, keepdims=True))
    loss = (jnp.sum(lse) - jnp.sum(v)) * (1.0 / _B)
    out_ref[...] = jnp.reshape(loss, (1, 1))


def _tc_loss(x, wg, t_col, t_row, interpret=False):
    return pl.pallas_call(
        _tc_loss_body,
        out_shape=jax.ShapeDtypeStruct((1, 1), jnp.float32),
        interpret=interpret,
    )(x, wg, t_col, t_row)


def kernel(input, target, W):
    w_flat = W.reshape(-1)
    wg = _sc_gather(w_flat, target.reshape(_B // 128, 128))
    t_col = target.reshape(_B, 1)
    t_row = target.reshape(1, _B)
    loss = _tc_loss(input, wg, t_col, t_row)
    return loss[0, 0]
